# baseline (speedup 1.0000x reference)
import functools

import jax
import jax.numpy as jnp
import numpy as np
from jax import lax
from jax.experimental import pallas as pl
from jax.experimental.pallas import tpu as pltpu
from jax.experimental.pallas import tpu_sc as plsc

N_NODES = 10000
IN_CH = 128
HID_CH = 128
FACTOR = 64
N_EDGES = 320000
BATCH = 4096

NC = 2
NS = 16
NW = NC * NS
LANES = 16

CH = 64
NCHUNK = 160
NPHASE = 4
NCHP = NCHUNK // NPHASE
EPT = NCHUNK * CH
EPAD = NW * EPT
NPAD = 10112
RPT = NPAD // NS
BPT = BATCH // NW

def _mesh():
    return plsc.VectorSubcoreMesh(
        core_axis_name="c", subcore_axis_name="s", num_cores=NC, num_subcores=NS
    )


@functools.partial(
    pl.kernel,
    out_type=jax.ShapeDtypeStruct((NW * NPAD,), jnp.float32),
    mesh=_mesh(),
    scratch_types=[
        pltpu.VMEM((EPT,), jnp.int32),
        pltpu.VMEM((NPAD,), jnp.float32),
    ],
    compiler_params=pltpu.CompilerParams(
        needs_layout_passes=False, use_tc_tiling_on_sc=False
    ),
)
def _deg_kernel(dsts_hbm, out_hbm, dst_v, hist):
    c = lax.axis_index("c")
    s = lax.axis_index("s")
    wid = s * NC + c
    pltpu.sync_copy(dsts_hbm.at[pl.ds(wid * EPT, EPT)], dst_v)

    def zbody(k, carry):
        hist[pl.ds(k * LANES, LANES)] = jnp.zeros((LANES,), jnp.float32)
        return carry

    lax.fori_loop(0, NPAD // LANES, zbody, 0)
    ones = jnp.ones((LANES,), jnp.float32)

    def body(j, carry):
        for k in range(4):
            idx = dst_v[pl.ds((j * 4 + k) * LANES, LANES)]
            plsc.addupdate_scatter(hist, [idx], ones)
        return carry

    lax.fori_loop(0, EPT // (4 * LANES), body, 0)
    pltpu.sync_copy(hist, out_hbm.at[pl.ds(wid * NPAD, NPAD)])


def _make_scatter(width, ch, nbuf, tc_tiling=True):
    nchp = EPT // ch // NPHASE
    epp = EPT // NPHASE
    @functools.partial(
        pl.kernel,
        out_type=jax.ShapeDtypeStruct((NC, NPAD, width), jnp.float32),
        mesh=_mesh(),
        compiler_params=pltpu.CompilerParams(use_tc_tiling_on_sc=tc_tiling),
        scratch_types=(
            [
                pltpu.VMEM((epp,), jnp.int32),
                pltpu.VMEM((epp,), jnp.int32),
                pltpu.VMEM_SHARED((NPAD, width), jnp.float32),
            ]
            + [pltpu.VMEM((ch, width), jnp.float32)] * nbuf
            + [pltpu.SemaphoreType.DMA] * nbuf
        ),
    )
    def _scatter_kernel(y_hbm, srcs_hbm, dsts_hbm, zeros_hbm, out_hbm,
                        src_v, dst_v, acc, *bufs_sems):
        rowbufs = bufs_sems[:nbuf]
        sems = bufs_sems[nbuf:]
        c = lax.axis_index("c")
        s = lax.axis_index("s")
        wid = s * NC + c
        r0 = s * RPT
        pltpu.sync_copy(zeros_hbm, acc.at[pl.ds(r0, RPT)])
        plsc.subcore_barrier()

        for phase in range(NPHASE):
            e0 = wid * EPT + phase * epp
            pltpu.sync_copy(srcs_hbm.at[pl.ds(e0, epp)], src_v)
            pltpu.sync_copy(dsts_hbm.at[pl.ds(e0, epp)], dst_v)
            for b in range(nbuf):
                pltpu.async_copy(y_hbm.at[src_v.at[pl.ds(b * ch, ch)]],
                                 rowbufs[b], sems[b])

            def body(g, carry):
                for b in range(nbuf):
                    j = g * nbuf + b
                    pltpu.make_async_copy(
                        y_hbm.at[src_v.at[pl.ds(j * ch, ch)]], rowbufs[b],
                        sems[b]).wait()
                    pltpu.sync_copy(rowbufs[b],
                                    acc.at[dst_v.at[pl.ds(j * ch, ch)]],
                                    add=True)
                    jn = j + nbuf

                    @pl.when(jn < nchp)
                    def _():
                        pltpu.async_copy(
                            y_hbm.at[src_v.at[pl.ds(jn * ch, ch)]],
                            rowbufs[b], sems[b])

                return carry

            lax.fori_loop(0, nchp // nbuf, body, 0)

        plsc.subcore_barrier()
        pltpu.sync_copy(acc.at[pl.ds(r0, RPT)], out_hbm.at[c, pl.ds(r0, RPT)])

    return _scatter_kernel


_scatter128 = _make_scatter(HID_CH, ch=128, nbuf=2, tc_tiling=False)
_scatter64 = _make_scatter(FACTOR, ch=128, nbuf=4, tc_tiling=False)


@functools.partial(
    pl.kernel,
    out_type=jax.ShapeDtypeStruct((2, BATCH), jnp.float32),
    mesh=_mesh(),
    compiler_params=pltpu.CompilerParams(
        use_tc_tiling_on_sc=False, needs_layout_passes=False
    ),
    scratch_types=[
        pltpu.VMEM((BPT,), jnp.int32),
        pltpu.VMEM((BPT, FACTOR), jnp.float32),
        pltpu.VMEM((BPT, FACTOR), jnp.float32),
        pltpu.VMEM((BPT, FACTOR), jnp.float32),
        pltpu.VMEM((BPT,), jnp.float32),
        pltpu.VMEM((BPT,), jnp.float32),
        pltpu.SemaphoreType.DMA,
    ],
)
def _gather_kernel(emb_hbm, ids0_hbm, ids1_hbm, ids2_hbm, out_hbm,
                   idx_v, bufa, bufi, bufj, predi, predj, sem):
    c = lax.axis_index("c")
    s = lax.axis_index("s")
    wid = s * NC + c
    base = wid * BPT
    for ids_hbm, buf in ((ids0_hbm, bufa), (ids1_hbm, bufi), (ids2_hbm, bufj)):
        pltpu.sync_copy(ids_hbm.at[pl.ds(base, BPT)], idx_v)
        pltpu.async_copy(emb_hbm.at[idx_v], buf, sem).wait()

    lane0 = lax.iota(jnp.int32, LANES) == 0

    def body(r, carry):
        acc_i = jnp.zeros((LANES,), jnp.float32)
        acc_j = jnp.zeros((LANES,), jnp.float32)
        for cb in range(FACTOR // LANES):
            va = bufa[r, pl.ds(cb * LANES, LANES)]
            acc_i = acc_i + va * bufi[r, pl.ds(cb * LANES, LANES)]
            acc_j = acc_j + va * bufj[r, pl.ds(cb * LANES, LANES)]
        ridx = jnp.full((LANES,), r, jnp.int32)
        plsc.store_scatter(predi, [ridx], jnp.full((LANES,), jnp.sum(acc_i)),
                           mask=lane0)
        plsc.store_scatter(predj, [ridx], jnp.full((LANES,), jnp.sum(acc_j)),
                           mask=lane0)
        return carry

    lax.fori_loop(0, BPT, body, 0)
    pltpu.sync_copy(predi, out_hbm.at[0, pl.ds(base, BPT)])
    pltpu.sync_copy(predj, out_hbm.at[1, pl.ds(base, BPT)])


def _dinv(deg_ref):
    total = deg_ref[pl.ds(0, NPAD)]
    for w in range(1, NW):
        total = total + deg_ref[pl.ds(w * NPAD, NPAD)]
    return lax.rsqrt(total + 1.0)[:, None]


def _b_body(x_ref, w_ref, deg_ref, y_ref):
    h = jnp.dot(x_ref[...], w_ref[...], preferred_element_type=jnp.float32)
    y_ref[pl.ds(0, N_NODES)] = h * _dinv(deg_ref)[:N_NODES]
    y_ref[pl.ds(N_NODES, NPAD - N_NODES)] = jnp.zeros(
        (NPAD - N_NODES, HID_CH), jnp.float32
    )


_tc_b = pl.pallas_call(
    _b_body, out_shape=jax.ShapeDtypeStruct((NPAD, HID_CH), jnp.float32)
)


def _d_body(p_ref, y1_ref, deg_ref, w2_ref, b1_ref, y2_ref):
    dinv = _dinv(deg_ref)
    z = p_ref[0] + p_ref[1] + y1_ref[...]
    h1 = jnp.maximum(z * dinv + b1_ref[...], 0.0)
    y2_ref[...] = jnp.dot(h1, w2_ref[...], preferred_element_type=jnp.float32) * dinv


_tc_d = pl.pallas_call(
    _d_body, out_shape=jax.ShapeDtypeStruct((NPAD, FACTOR), jnp.float32)
)


def _f_body(q_ref, y2_ref, deg_ref, b2_ref, emb_ref, node_ref):
    z2 = q_ref[0] + q_ref[1] + y2_ref[...]
    e = z2 * _dinv(deg_ref) + b2_ref[...]
    emb_ref[...] = e
    node_ref[...] = e[:N_NODES]


_tc_f = pl.pallas_call(
    _f_body,
    out_shape=[
        jax.ShapeDtypeStruct((NPAD, FACTOR), jnp.float32),
        jax.ShapeDtypeStruct((N_NODES, FACTOR), jnp.float32),
    ],
)


_PAD_IDX = np.asarray(
    N_NODES + np.arange(EPAD - N_EDGES) % (NPAD - N_NODES), dtype=np.int32
)


def kernel(drug_ids, disease_ids_i, disease_ids_j, x, edge_index, W1, b1, W2, b2):
    src = edge_index[0].astype(jnp.int32)
    dst = edge_index[1].astype(jnp.int32)
    pad = jnp.asarray(_PAD_IDX)
    srcs = jnp.concatenate([src, pad])
    dsts = jnp.concatenate([dst, pad])

    degp = _deg_kernel(dsts)

    y1 = _tc_b(x, W1, degp)

    p = _scatter128(y1, srcs, dsts, jnp.zeros((RPT, HID_CH), jnp.float32))
    y2 = _tc_d(p, y1, degp, W2, b1)

    q = _scatter64(y2, srcs, dsts, jnp.zeros((RPT, FACTOR), jnp.float32))
    emb_p, node_emb = _tc_f(q, y2, degp, b2)

    preds = _gather_kernel(
        emb_p,
        drug_ids.astype(jnp.int32),
        disease_ids_i.astype(jnp.int32),
        disease_ids_j.astype(jnp.int32),
    )
    return (preds[0], preds[1], node_emb)


# speedup vs baseline: 1.0526x; 1.0526x over previous
import functools

import jax
import jax.numpy as jnp
import numpy as np
from jax import lax
from jax.experimental import pallas as pl
from jax.experimental.pallas import tpu as pltpu
from jax.experimental.pallas import tpu_sc as plsc

N_NODES = 10000
IN_CH = 128
HID_CH = 128
FACTOR = 64
N_EDGES = 320000
BATCH = 4096

NC = 2
NS = 16
NW = NC * NS
LANES = 16

CH = 64
NCHUNK = 160
NPHASE = 4
NCHP = NCHUNK // NPHASE
EPT = NCHUNK * CH
EPAD = NW * EPT
NPAD = 10112
RPT = NPAD // NS
BPT = BATCH // NW

def _mesh():
    return plsc.VectorSubcoreMesh(
        core_axis_name="c", subcore_axis_name="s", num_cores=NC, num_subcores=NS
    )


@functools.partial(
    pl.kernel,
    out_type=jax.ShapeDtypeStruct((NW * NPAD,), jnp.float32),
    mesh=_mesh(),
    scratch_types=[
        pltpu.VMEM((EPT,), jnp.int32),
        pltpu.VMEM((NPAD,), jnp.float32),
    ],
    compiler_params=pltpu.CompilerParams(
        needs_layout_passes=False, use_tc_tiling_on_sc=False
    ),
)
def _deg_kernel(dsts_hbm, out_hbm, dst_v, hist):
    c = lax.axis_index("c")
    s = lax.axis_index("s")
    wid = s * NC + c
    pltpu.sync_copy(dsts_hbm.at[pl.ds(wid * EPT, EPT)], dst_v)

    def zbody(k, carry):
        hist[pl.ds(k * LANES, LANES)] = jnp.zeros((LANES,), jnp.float32)
        return carry

    lax.fori_loop(0, NPAD // LANES, zbody, 0)
    ones = jnp.ones((LANES,), jnp.float32)

    def body(j, carry):
        for k in range(4):
            idx = dst_v[pl.ds((j * 4 + k) * LANES, LANES)]
            plsc.addupdate_scatter(hist, [idx], ones)
        return carry

    lax.fori_loop(0, EPT // (4 * LANES), body, 0)
    pltpu.sync_copy(hist, out_hbm.at[pl.ds(wid * NPAD, NPAD)])


def _make_scatter(width, ch, nbuf, tc_tiling=True):
    nchp = EPT // ch // NPHASE
    epp = EPT // NPHASE
    @functools.partial(
        pl.kernel,
        out_type=jax.ShapeDtypeStruct((NC, NPAD, width), jnp.float32),
        mesh=_mesh(),
        compiler_params=pltpu.CompilerParams(use_tc_tiling_on_sc=tc_tiling),
        scratch_types=(
            [
                pltpu.VMEM((epp,), jnp.int32),
                pltpu.VMEM((epp,), jnp.int32),
                pltpu.VMEM_SHARED((NPAD, width), jnp.float32),
            ]
            + [pltpu.VMEM((ch, width), jnp.float32)] * nbuf
            + [pltpu.SemaphoreType.DMA] * nbuf
        ),
    )
    def _scatter_kernel(y_hbm, srcs_hbm, dsts_hbm, zeros_hbm, out_hbm,
                        src_v, dst_v, acc, *bufs_sems):
        rowbufs = bufs_sems[:nbuf]
        sems = bufs_sems[nbuf:]
        c = lax.axis_index("c")
        s = lax.axis_index("s")
        wid = s * NC + c
        r0 = s * RPT
        pltpu.sync_copy(zeros_hbm, acc.at[pl.ds(r0, RPT)])
        plsc.subcore_barrier()

        for phase in range(NPHASE):
            e0 = wid * EPT + phase * epp
            pltpu.sync_copy(srcs_hbm.at[pl.ds(e0, epp)], src_v)
            pltpu.sync_copy(dsts_hbm.at[pl.ds(e0, epp)], dst_v)
            for b in range(nbuf):
                pltpu.async_copy(y_hbm.at[src_v.at[pl.ds(b * ch, ch)]],
                                 rowbufs[b], sems[b])

            def body(g, carry):
                for b in range(nbuf):
                    j = g * nbuf + b
                    pltpu.make_async_copy(
                        y_hbm.at[src_v.at[pl.ds(j * ch, ch)]], rowbufs[b],
                        sems[b]).wait()
                    pltpu.sync_copy(rowbufs[b],
                                    acc.at[dst_v.at[pl.ds(j * ch, ch)]],
                                    add=True)
                    jn = j + nbuf

                    @pl.when(jn < nchp)
                    def _():
                        pltpu.async_copy(
                            y_hbm.at[src_v.at[pl.ds(jn * ch, ch)]],
                            rowbufs[b], sems[b])

                return carry

            lax.fori_loop(0, nchp // nbuf, body, 0)

        plsc.subcore_barrier()
        pltpu.sync_copy(acc.at[pl.ds(r0, RPT)], out_hbm.at[c, pl.ds(r0, RPT)])

    return _scatter_kernel


_scatter128 = _make_scatter(HID_CH, ch=64, nbuf=4, tc_tiling=False)
_scatter64 = _make_scatter(FACTOR, ch=128, nbuf=4, tc_tiling=False)


@functools.partial(
    pl.kernel,
    out_type=jax.ShapeDtypeStruct((2, BATCH), jnp.float32),
    mesh=_mesh(),
    compiler_params=pltpu.CompilerParams(
        use_tc_tiling_on_sc=False, needs_layout_passes=False
    ),
    scratch_types=[
        pltpu.VMEM((BPT,), jnp.int32),
        pltpu.VMEM((BPT, FACTOR), jnp.float32),
        pltpu.VMEM((BPT, FACTOR), jnp.float32),
        pltpu.VMEM((BPT, FACTOR), jnp.float32),
        pltpu.VMEM((BPT,), jnp.float32),
        pltpu.VMEM((BPT,), jnp.float32),
        pltpu.SemaphoreType.DMA,
    ],
)
def _gather_kernel(emb_hbm, ids0_hbm, ids1_hbm, ids2_hbm, out_hbm,
                   idx_v, bufa, bufi, bufj, predi, predj, sem):
    c = lax.axis_index("c")
    s = lax.axis_index("s")
    wid = s * NC + c
    base = wid * BPT
    for ids_hbm, buf in ((ids0_hbm, bufa), (ids1_hbm, bufi), (ids2_hbm, bufj)):
        pltpu.sync_copy(ids_hbm.at[pl.ds(base, BPT)], idx_v)
        pltpu.async_copy(emb_hbm.at[idx_v], buf, sem).wait()

    lane0 = lax.iota(jnp.int32, LANES) == 0

    def body(r, carry):
        acc_i = jnp.zeros((LANES,), jnp.float32)
        acc_j = jnp.zeros((LANES,), jnp.float32)
        for cb in range(FACTOR // LANES):
            va = bufa[r, pl.ds(cb * LANES, LANES)]
            acc_i = acc_i + va * bufi[r, pl.ds(cb * LANES, LANES)]
            acc_j = acc_j + va * bufj[r, pl.ds(cb * LANES, LANES)]
        ridx = jnp.full((LANES,), r, jnp.int32)
        plsc.store_scatter(predi, [ridx], jnp.full((LANES,), jnp.sum(acc_i)),
                           mask=lane0)
        plsc.store_scatter(predj, [ridx], jnp.full((LANES,), jnp.sum(acc_j)),
                           mask=lane0)
        return carry

    lax.fori_loop(0, BPT, body, 0)
    pltpu.sync_copy(predi, out_hbm.at[0, pl.ds(base, BPT)])
    pltpu.sync_copy(predj, out_hbm.at[1, pl.ds(base, BPT)])


def _dinv(deg_ref):
    total = deg_ref[pl.ds(0, NPAD)]
    for w in range(1, NW):
        total = total + deg_ref[pl.ds(w * NPAD, NPAD)]
    return lax.rsqrt(total + 1.0)[:, None]


def _b_body(x_ref, w_ref, deg_ref, y_ref):
    h = jnp.dot(x_ref[...], w_ref[...], preferred_element_type=jnp.float32)
    y_ref[pl.ds(0, N_NODES)] = h * _dinv(deg_ref)[:N_NODES]
    y_ref[pl.ds(N_NODES, NPAD - N_NODES)] = jnp.zeros(
        (NPAD - N_NODES, HID_CH), jnp.float32
    )


_tc_b = pl.pallas_call(
    _b_body, out_shape=jax.ShapeDtypeStruct((NPAD, HID_CH), jnp.float32)
)


def _d_body(p_ref, y1_ref, deg_ref, w2_ref, b1_ref, y2_ref):
    dinv = _dinv(deg_ref)
    z = p_ref[0] + p_ref[1] + y1_ref[...]
    h1 = jnp.maximum(z * dinv + b1_ref[...], 0.0)
    y2_ref[...] = jnp.dot(h1, w2_ref[...], preferred_element_type=jnp.float32) * dinv


_tc_d = pl.pallas_call(
    _d_body, out_shape=jax.ShapeDtypeStruct((NPAD, FACTOR), jnp.float32)
)


def _f_body(q_ref, y2_ref, deg_ref, b2_ref, emb_ref, node_ref):
    z2 = q_ref[0] + q_ref[1] + y2_ref[...]
    e = z2 * _dinv(deg_ref) + b2_ref[...]
    emb_ref[...] = e
    node_ref[...] = e[:N_NODES]


_tc_f = pl.pallas_call(
    _f_body,
    out_shape=[
        jax.ShapeDtypeStruct((NPAD, FACTOR), jnp.float32),
        jax.ShapeDtypeStruct((N_NODES, FACTOR), jnp.float32),
    ],
)


_PAD_IDX = np.asarray(
    N_NODES + np.arange(EPAD - N_EDGES) % (NPAD - N_NODES), dtype=np.int32
)


def kernel(drug_ids, disease_ids_i, disease_ids_j, x, edge_index, W1, b1, W2, b2):
    src = edge_index[0].astype(jnp.int32)
    dst = edge_index[1].astype(jnp.int32)
    pad = jnp.asarray(_PAD_IDX)
    srcs = jnp.concatenate([src, pad])
    dsts = jnp.concatenate([dst, pad])

    degp = _deg_kernel(dsts)

    y1 = _tc_b(x, W1, degp)

    p = _scatter128(y1, srcs, dsts, jnp.zeros((RPT, HID_CH), jnp.float32))
    y2 = _tc_d(p, y1, degp, W2, b1)

    q = _scatter64(y2, srcs, dsts, jnp.zeros((RPT, FACTOR), jnp.float32))
    emb_p, node_emb = _tc_f(q, y2, degp, b2)

    preds = _gather_kernel(
        emb_p,
        drug_ids.astype(jnp.int32),
        disease_ids_i.astype(jnp.int32),
        disease_ids_j.astype(jnp.int32),
    )
    return (preds[0], preds[1], node_emb)


# speedup vs baseline: 1.0832x; 1.0290x over previous
import functools

import jax
import jax.numpy as jnp
import numpy as np
from jax import lax
from jax.experimental import pallas as pl
from jax.experimental.pallas import tpu as pltpu
from jax.experimental.pallas import tpu_sc as plsc

N_NODES = 10000
IN_CH = 128
HID_CH = 128
FACTOR = 64
N_EDGES = 320000
BATCH = 4096

NC = 2
NS = 16
NW = NC * NS
LANES = 16

CH = 64
NCHUNK = 160
NPHASE = 4
NCHP = NCHUNK // NPHASE
EPT = NCHUNK * CH
EPAD = NW * EPT
NPAD = 10112
RPT = NPAD // NS
BPT = BATCH // NW

def _mesh():
    return plsc.VectorSubcoreMesh(
        core_axis_name="c", subcore_axis_name="s", num_cores=NC, num_subcores=NS
    )


@functools.partial(
    pl.kernel,
    out_type=jax.ShapeDtypeStruct((NW * NPAD,), jnp.float32),
    mesh=_mesh(),
    scratch_types=[
        pltpu.VMEM((EPT,), jnp.int32),
        pltpu.VMEM((NPAD,), jnp.float32),
    ],
    compiler_params=pltpu.CompilerParams(
        needs_layout_passes=False, use_tc_tiling_on_sc=False
    ),
)
def _deg_kernel(dsts_hbm, out_hbm, dst_v, hist):
    c = lax.axis_index("c")
    s = lax.axis_index("s")
    wid = s * NC + c
    pltpu.sync_copy(dsts_hbm.at[pl.ds(wid * EPT, EPT)], dst_v)

    def zbody(k, carry):
        hist[pl.ds(k * LANES, LANES)] = jnp.zeros((LANES,), jnp.float32)
        return carry

    lax.fori_loop(0, NPAD // LANES, zbody, 0)
    ones = jnp.ones((LANES,), jnp.float32)

    def body(j, carry):
        for k in range(4):
            idx = dst_v[pl.ds((j * 4 + k) * LANES, LANES)]
            plsc.addupdate_scatter(hist, [idx], ones)
        return carry

    lax.fori_loop(0, EPT // (4 * LANES), body, 0)
    pltpu.sync_copy(hist, out_hbm.at[pl.ds(wid * NPAD, NPAD)])


def _make_scatter(width, ch, nbuf, tc_tiling=True, split_out=False):
    nchp = EPT // ch // NPHASE
    epp = EPT // NPHASE
    if split_out:
        out_type = [jax.ShapeDtypeStruct((NPAD, width), jnp.float32)] * 2
    else:
        out_type = jax.ShapeDtypeStruct((NC, NPAD, width), jnp.float32)

    @functools.partial(
        pl.kernel,
        out_type=out_type,
        mesh=_mesh(),
        compiler_params=pltpu.CompilerParams(use_tc_tiling_on_sc=tc_tiling),
        scratch_types=(
            [
                pltpu.VMEM((epp,), jnp.int32),
                pltpu.VMEM((epp,), jnp.int32),
                pltpu.VMEM_SHARED((NPAD, width), jnp.float32),
            ]
            + [pltpu.VMEM((ch, width), jnp.float32)] * nbuf
            + [pltpu.SemaphoreType.DMA] * nbuf
        ),
    )
    def _scatter_kernel(y_hbm, srcs_hbm, dsts_hbm, zeros_hbm, *out_and_scratch):
        if split_out:
            out0_hbm, out1_hbm, src_v, dst_v, acc, *bufs_sems = out_and_scratch
        else:
            out_hbm, src_v, dst_v, acc, *bufs_sems = out_and_scratch
        rowbufs = bufs_sems[:nbuf]
        sems = bufs_sems[nbuf:]
        c = lax.axis_index("c")
        s = lax.axis_index("s")
        wid = s * NC + c
        r0 = s * RPT
        pltpu.sync_copy(zeros_hbm, acc.at[pl.ds(r0, RPT)])
        plsc.subcore_barrier()

        for phase in range(NPHASE):
            e0 = wid * EPT + phase * epp
            pltpu.sync_copy(srcs_hbm.at[pl.ds(e0, epp)], src_v)
            pltpu.sync_copy(dsts_hbm.at[pl.ds(e0, epp)], dst_v)
            for b in range(nbuf):
                pltpu.async_copy(y_hbm.at[src_v.at[pl.ds(b * ch, ch)]],
                                 rowbufs[b], sems[b])

            def body(g, carry):
                for b in range(nbuf):
                    j = g * nbuf + b
                    pltpu.make_async_copy(
                        y_hbm.at[src_v.at[pl.ds(j * ch, ch)]], rowbufs[b],
                        sems[b]).wait()
                    pltpu.sync_copy(rowbufs[b],
                                    acc.at[dst_v.at[pl.ds(j * ch, ch)]],
                                    add=True)
                    jn = j + nbuf

                    @pl.when(jn < nchp)
                    def _():
                        pltpu.async_copy(
                            y_hbm.at[src_v.at[pl.ds(jn * ch, ch)]],
                            rowbufs[b], sems[b])

                return carry

            lax.fori_loop(0, nchp // nbuf, body, 0)

        plsc.subcore_barrier()
        if split_out:
            @pl.when(c == 0)
            def _():
                pltpu.sync_copy(acc.at[pl.ds(r0, RPT)],
                                out0_hbm.at[pl.ds(r0, RPT)])

            @pl.when(c == 1)
            def _():
                pltpu.sync_copy(acc.at[pl.ds(r0, RPT)],
                                out1_hbm.at[pl.ds(r0, RPT)])
        else:
            pltpu.sync_copy(acc.at[pl.ds(r0, RPT)],
                            out_hbm.at[c, pl.ds(r0, RPT)])

    return _scatter_kernel


_scatter128 = _make_scatter(HID_CH, ch=64, nbuf=4, tc_tiling=False)
_scatter64 = _make_scatter(FACTOR, ch=128, nbuf=4, tc_tiling=False,
                           split_out=True)


@functools.partial(
    pl.kernel,
    out_type=jax.ShapeDtypeStruct((2, BATCH), jnp.float32),
    mesh=_mesh(),
    compiler_params=pltpu.CompilerParams(
        use_tc_tiling_on_sc=False, needs_layout_passes=False
    ),
    scratch_types=(
        [pltpu.VMEM((BPT,), jnp.int32)] * 3
        + [pltpu.VMEM((BPT, FACTOR), jnp.float32)] * 9
        + [pltpu.VMEM((NPAD,), jnp.float32), pltpu.VMEM((FACTOR,), jnp.float32)]
        + [pltpu.VMEM((BPT,), jnp.float32)] * 7
        + [pltpu.SemaphoreType.DMA]
    ),
)
def _bpr_kernel(q0_hbm, q1_hbm, y2_hbm, dinv_hbm, b2_hbm,
                ids0_hbm, ids1_hbm, ids2_hbm, out_hbm,
                idx0, idx1, idx2, *scratch):
    bufs = scratch[:9]
    dinv_v, b2_v = scratch[9:11]
    t_abi, t_abj, t_a, t_bi, t_bj, predi, predj = scratch[11:18]
    sem = scratch[18]
    c = lax.axis_index("c")
    s = lax.axis_index("s")
    wid = s * NC + c
    base = wid * BPT
    pltpu.sync_copy(ids0_hbm.at[pl.ds(base, BPT)], idx0)
    pltpu.sync_copy(ids1_hbm.at[pl.ds(base, BPT)], idx1)
    pltpu.sync_copy(ids2_hbm.at[pl.ds(base, BPT)], idx2)
    pltpu.sync_copy(dinv_hbm, dinv_v)
    pltpu.sync_copy(b2_hbm, b2_v)
    copies = []
    for t, idx in enumerate((idx0, idx1, idx2)):
        for u, tab in enumerate((q0_hbm, q1_hbm, y2_hbm)):
            copies.append(pltpu.async_copy(tab.at[idx], bufs[t * 3 + u], sem))
    for cp in copies:
        cp.wait()

    lane0 = lax.iota(jnp.int32, LANES) == 0
    b2s = [b2_v[pl.ds(cb * LANES, LANES)] for cb in range(FACTOR // LANES)]
    b2sq = sum(jnp.sum(v * v) for v in b2s)

    def body(r, carry):
        zl = jnp.zeros((LANES,), jnp.float32)
        abi = abj = ta = tbi = tbj = zl
        for cb in range(FACTOR // LANES):
            sl = pl.ds(cb * LANES, LANES)
            sa = bufs[0][r, sl] + bufs[1][r, sl] + bufs[2][r, sl]
            si = bufs[3][r, sl] + bufs[4][r, sl] + bufs[5][r, sl]
            sj = bufs[6][r, sl] + bufs[7][r, sl] + bufs[8][r, sl]
            abi = abi + sa * si
            abj = abj + sa * sj
            ta = ta + sa * b2s[cb]
            tbi = tbi + si * b2s[cb]
            tbj = tbj + sj * b2s[cb]
        ridx = jnp.full((LANES,), r, jnp.int32)
        for ref, acc in ((t_abi, abi), (t_abj, abj), (t_a, ta),
                         (t_bi, tbi), (t_bj, tbj)):
            plsc.store_scatter(ref, [ridx], jnp.full((LANES,), jnp.sum(acc)),
                               mask=lane0)
        return carry

    lax.fori_loop(0, BPT, body, 0)

    for g in range(BPT // LANES):
        sl = pl.ds(g * LANES, LANES)
        da = plsc.load_gather(dinv_v, [idx0[sl]])
        di = plsc.load_gather(dinv_v, [idx1[sl]])
        dj = plsc.load_gather(dinv_v, [idx2[sl]])
        predi[sl] = da * di * t_abi[sl] + da * t_a[sl] + di * t_bi[sl] + b2sq
        predj[sl] = da * dj * t_abj[sl] + da * t_a[sl] + dj * t_bj[sl] + b2sq

    pltpu.sync_copy(predi, out_hbm.at[0, pl.ds(base, BPT)])
    pltpu.sync_copy(predj, out_hbm.at[1, pl.ds(base, BPT)])


def _dinv(deg_ref):
    total = deg_ref[pl.ds(0, NPAD)]
    for w in range(1, NW):
        total = total + deg_ref[pl.ds(w * NPAD, NPAD)]
    return lax.rsqrt(total + 1.0)[:, None]


def _b_body(x_ref, w_ref, deg_ref, y_ref):
    h = jnp.dot(x_ref[...], w_ref[...], preferred_element_type=jnp.float32)
    y_ref[pl.ds(0, N_NODES)] = h * _dinv(deg_ref)[:N_NODES]
    y_ref[pl.ds(N_NODES, NPAD - N_NODES)] = jnp.zeros(
        (NPAD - N_NODES, HID_CH), jnp.float32
    )


_tc_b = pl.pallas_call(
    _b_body, out_shape=jax.ShapeDtypeStruct((NPAD, HID_CH), jnp.float32)
)


def _d_body(p_ref, y1_ref, deg_ref, w2_ref, b1_ref, y2_ref, dinv_ref):
    total = deg_ref[pl.ds(0, NPAD)]
    for w in range(1, NW):
        total = total + deg_ref[pl.ds(w * NPAD, NPAD)]
    dv = lax.rsqrt(total + 1.0)
    dinv = dv[:, None]
    z = p_ref[0] + p_ref[1] + y1_ref[...]
    h1 = jnp.maximum(z * dinv + b1_ref[...], 0.0)
    y2_ref[...] = jnp.dot(h1, w2_ref[...], preferred_element_type=jnp.float32) * dinv
    dinv_ref[...] = dv


_tc_d = pl.pallas_call(
    _d_body,
    out_shape=[
        jax.ShapeDtypeStruct((NPAD, FACTOR), jnp.float32),
        jax.ShapeDtypeStruct((NPAD,), jnp.float32),
    ],
)


def _f_body(q0_ref, q1_ref, y2_ref, deg_ref, b2_ref, node_ref):
    z2 = q0_ref[...] + q1_ref[...] + y2_ref[...]
    e = z2 * _dinv(deg_ref) + b2_ref[...]
    node_ref[...] = e[:N_NODES]


_tc_f = pl.pallas_call(
    _f_body, out_shape=jax.ShapeDtypeStruct((N_NODES, FACTOR), jnp.float32)
)


_PAD_IDX = np.asarray(
    N_NODES + np.arange(EPAD - N_EDGES) % (NPAD - N_NODES), dtype=np.int32
)


def kernel(drug_ids, disease_ids_i, disease_ids_j, x, edge_index, W1, b1, W2, b2):
    src = edge_index[0].astype(jnp.int32)
    dst = edge_index[1].astype(jnp.int32)
    pad = jnp.asarray(_PAD_IDX)
    srcs = jnp.concatenate([src, pad])
    dsts = jnp.concatenate([dst, pad])

    degp = _deg_kernel(dsts)

    y1 = _tc_b(x, W1, degp)

    p = _scatter128(y1, srcs, dsts, jnp.zeros((RPT, HID_CH), jnp.float32))
    y2, dinv1 = _tc_d(p, y1, degp, W2, b1)

    q0, q1 = _scatter64(y2, srcs, dsts, jnp.zeros((RPT, FACTOR), jnp.float32))
    node_emb = _tc_f(q0, q1, y2, degp, b2)
    preds = _bpr_kernel(
        q0, q1, y2, dinv1, b2,
        drug_ids.astype(jnp.int32),
        disease_ids_i.astype(jnp.int32),
        disease_ids_j.astype(jnp.int32),
    )
    return (preds[0], preds[1], node_emb)
